# Initial kernel scaffold; baseline (speedup 1.0000x reference)
import jax
import jax.numpy as jnp
from jax.experimental import pallas as pl


def kernel(indices, amplitude_table, phase_table):
    raise NotImplementedError("write your pallas kernel here")


# speedup vs baseline: 6.2632x; 6.2632x over previous
import functools

import jax
import jax.numpy as jnp
from jax import lax
from jax.experimental import pallas as pl
from jax.experimental.pallas import tpu as pltpu
from jax.experimental.pallas import tpu_sc as plsc

EMBED_DIM = 64
CHUNK = 128


@functools.lru_cache(maxsize=None)
def _make_lookup(n_total: int, dim: int):
    info = plsc.get_sparse_core_info()
    num_cores, num_subcores = info.num_cores, info.num_subcores
    num_workers = num_cores * num_subcores
    assert n_total % (num_workers * CHUNK) == 0
    per_worker = n_total // num_workers
    n_chunks = per_worker // CHUNK

    mesh = plsc.VectorSubcoreMesh(core_axis_name="c", subcore_axis_name="s")

    @functools.partial(
        pl.kernel,
        mesh=mesh,
        out_type=(
            jax.ShapeDtypeStruct((n_total, dim), jnp.float32),
            jax.ShapeDtypeStruct((n_total, dim), jnp.float32),
        ),
        scratch_types=[
            pltpu.VMEM((n_chunks, CHUNK), jnp.int32),
            pltpu.VMEM((CHUNK, dim), jnp.float32),
            pltpu.VMEM((CHUNK, dim), jnp.float32),
            pltpu.SemaphoreType.DMA,
            pltpu.SemaphoreType.DMA,
        ],
        compiler_params=pltpu.CompilerParams(use_tc_tiling_on_sc=False),
    )
    def lookup(idx_hbm, amp_hbm, ph_hbm, amp_out, ph_out,
               idx_v, amp_v, ph_v, sem_a, sem_p):
        wid = lax.axis_index("s") * num_cores + lax.axis_index("c")
        base_w = pl.multiple_of(wid * per_worker, CHUNK)
        pltpu.sync_copy(
            idx_hbm.at[pl.ds(pl.multiple_of(wid * n_chunks, 8), n_chunks)],
            idx_v)

        def body(i, carry):
            base = pl.multiple_of(base_w + i * CHUNK, CHUNK)
            idx_row = idx_v.at[i]
            ca = pltpu.async_copy(amp_hbm.at[idx_row], amp_v, sem_a)
            cp = pltpu.async_copy(ph_hbm.at[idx_row], ph_v, sem_p)
            ca.wait()
            cp.wait()
            pltpu.sync_copy(amp_v, amp_out.at[pl.ds(base, CHUNK)])
            pltpu.sync_copy(ph_v, ph_out.at[pl.ds(base, CHUNK)])
            return carry

        lax.fori_loop(0, n_chunks, body, 0)

    return lookup


def kernel(indices, amplitude_table, phase_table):
    batch, hist = indices.shape
    n_total = batch * hist
    dim = amplitude_table.shape[1]
    flat_idx = indices.reshape(n_total // CHUNK, CHUNK)
    lookup = _make_lookup(n_total, dim)
    amp, ph = lookup(flat_idx, amplitude_table, phase_table)
    return amp.reshape(batch, hist, dim), ph.reshape(batch, hist, dim)


# speedup vs baseline: 6.8271x; 1.0900x over previous
import functools

import jax
import jax.numpy as jnp
from jax import lax
from jax.experimental import pallas as pl
from jax.experimental.pallas import tpu as pltpu
from jax.experimental.pallas import tpu_sc as plsc

EMBED_DIM = 64
CHUNK = 128
N_BUF = 3


@functools.lru_cache(maxsize=None)
def _make_lookup(n_total: int, dim: int):
    info = plsc.get_sparse_core_info()
    num_cores, num_subcores = info.num_cores, info.num_subcores
    num_workers = num_cores * num_subcores
    assert n_total % (num_workers * CHUNK) == 0
    per_worker = n_total // num_workers
    n_chunks = per_worker // CHUNK
    assert n_chunks >= 6 and (n_chunks - 5) % 3 == 0
    n_groups = (n_chunks - 5) // 3

    mesh = plsc.VectorSubcoreMesh(core_axis_name="c", subcore_axis_name="s")

    @functools.partial(
        pl.kernel,
        mesh=mesh,
        out_type=(
            jax.ShapeDtypeStruct((n_total, dim), jnp.float32),
            jax.ShapeDtypeStruct((n_total, dim), jnp.float32),
        ),
        scratch_types=[
            pltpu.VMEM((n_chunks, CHUNK), jnp.int32),
            [pltpu.VMEM((CHUNK, dim), jnp.float32)] * N_BUF,
            [pltpu.VMEM((CHUNK, dim), jnp.float32)] * N_BUF,
            [pltpu.SemaphoreType.DMA] * N_BUF,
            [pltpu.SemaphoreType.DMA] * N_BUF,
        ],
        compiler_params=pltpu.CompilerParams(use_tc_tiling_on_sc=False),
    )
    def lookup(idx_hbm, amp_hbm, ph_hbm, amp_out, ph_out,
               idx_v, amp_bufs, ph_bufs, sem_g, sem_w):
        wid = lax.axis_index("s") * num_cores + lax.axis_index("c")
        base_w = pl.multiple_of(wid * per_worker, CHUNK)
        pltpu.sync_copy(
            idx_hbm.at[pl.ds(pl.multiple_of(wid * n_chunks, 8), n_chunks)],
            idx_v)

        def start_g(j, b):
            row = idx_v.at[j]
            pltpu.async_copy(amp_hbm.at[row], amp_bufs[b], sem_g[b])
            pltpu.async_copy(ph_hbm.at[row], ph_bufs[b], sem_g[b])

        def wait_g(b):
            row = idx_v.at[0]
            pltpu.make_async_copy(amp_hbm.at[row], amp_bufs[b], sem_g[b]).wait()
            pltpu.make_async_copy(ph_hbm.at[row], ph_bufs[b], sem_g[b]).wait()

        def start_w(j, b):
            base = pl.multiple_of(base_w + j * CHUNK, CHUNK)
            pltpu.async_copy(amp_bufs[b], amp_out.at[pl.ds(base, CHUNK)], sem_w[b])
            pltpu.async_copy(ph_bufs[b], ph_out.at[pl.ds(base, CHUNK)], sem_w[b])

        def wait_w(b):
            dst = amp_out.at[pl.ds(0, CHUNK)]
            pltpu.make_async_copy(amp_bufs[b], dst, sem_w[b]).wait()
            pltpu.make_async_copy(ph_bufs[b], dst, sem_w[b]).wait()

        start_g(0, 0)
        start_g(1, 1)
        wait_g(0)
        start_w(0, 0)
        start_g(2, 2)
        wait_g(1)
        start_w(1, 1)
        wait_w(0)
        start_g(3, 0)
        wait_g(2)
        start_w(2, 2)
        wait_w(1)
        start_g(4, 1)
        wait_g(0)
        start_w(3, 0)

        def body(g, carry):
            for k in range(3):
                i = 4 + 3 * g + k
                b = (1 + k) % 3
                b_next = (2 + k) % 3
                wait_w(b_next)
                start_g(i + 1, b_next)
                wait_g(b)
                start_w(i, b)
            return carry

        lax.fori_loop(0, n_groups, body, 0)

        wait_g(1)
        start_w(n_chunks - 1, 1)
        wait_w(2)
        wait_w(0)
        wait_w(1)

    return lookup


def kernel(indices, amplitude_table, phase_table):
    batch, hist = indices.shape
    n_total = batch * hist
    dim = amplitude_table.shape[1]
    flat_idx = indices.reshape(n_total // CHUNK, CHUNK)
    lookup = _make_lookup(n_total, dim)
    amp, ph = lookup(flat_idx, amplitude_table, phase_table)
    return amp.reshape(batch, hist, dim), ph.reshape(batch, hist, dim)
